# baseline (speedup 1.0000x reference)
import functools
import math

import jax
import jax.numpy as jnp
import numpy as np
from jax.experimental import pallas as pl
from jax.experimental.pallas import tpu as pltpu

_FACTOR = 5
_PAD = 64
_HIGHEST = jax.lax.Precision.HIGHEST
_QB = 256
_HB = 8


def _threefry2x32(k0, k1, x0, x1):
    def rotl(x, d):
        return (x << np.uint32(d)) | (x >> np.uint32(32 - d))

    ks0, ks1 = np.uint32(k0), np.uint32(k1)
    ks2 = ks0 ^ ks1 ^ np.uint32(0x1BD11BDA)
    x0 = x0 + ks0
    x1 = x1 + ks1
    rot1, rot2 = (13, 15, 26, 6), (17, 29, 16, 24)
    sched = [(rot1, ks1, ks2, 1), (rot2, ks2, ks0, 2), (rot1, ks0, ks1, 3),
             (rot2, ks1, ks2, 4), (rot1, ks2, ks0, 5)]
    for rots, a, b, i in sched:
        for r in rots:
            x0 = x0 + x1
            x1 = rotl(x1, r)
            x1 = x1 ^ x0
        x0 = x0 + a
        x1 = x1 + b + np.uint32(i)
    return x0, x1


def _tf_random_bits(k, n):
    i = np.arange(n, dtype=np.uint64)
    hi = (i >> np.uint64(32)).astype(np.uint32)
    lo = (i & np.uint64(0xFFFFFFFF)).astype(np.uint32)
    b1, b2 = _threefry2x32(k[0], k[1], hi, lo)
    return b1 ^ b2


@functools.lru_cache(maxsize=None)
def _count_table(L_Q: int, L_K: int, sample_k: int):
    assert L_K & (L_K - 1) == 0
    old = np.seterr(over="ignore")
    try:
        key = (np.uint32(0), np.uint32(42))
        hi = np.zeros(2, np.uint32)
        lo = np.arange(2, dtype=np.uint32)
        b1, b2 = _threefry2x32(key[0], key[1], hi, lo)
        k2 = (b1[1], b2[1])
        bits = _tf_random_bits(k2, L_Q * sample_k).reshape(L_Q, sample_k)
        idx = (bits % np.uint32(L_K)).astype(np.int32)
    finally:
        np.seterr(**old)
    cnt = np.zeros((L_Q, L_K), np.float32)
    np.add.at(cnt, (np.arange(L_Q)[:, None], idx), 1.0)
    import ml_dtypes
    cnt_bf16 = cnt.astype(ml_dtypes.bfloat16)
    ninf = np.where(cnt > 0.0, 0.0, -np.inf).astype(np.float32)
    return cnt_bf16, ninf


def _mstats_body(n_top, q_ref, k_ref, cnt_ref, ninf_ref, pos_ref, post_ref,
                 m_acc):
    H = q_ref.shape[1]
    L_K = k_ref.shape[0]
    i = pl.program_id(0)
    cnt = cnt_ref[...]
    ninf = ninf_ref[...]
    for h in range(H):
        qh = q_ref[:, h, :]
        kh = k_ref[:, h, :]
        s = jax.lax.dot_general(
            qh, kh, (((1,), (1,)), ((), ())),
            preferred_element_type=jnp.float32)
        smax = jnp.max(s + ninf, axis=1)
        ks = jax.lax.dot_general(
            cnt, kh, (((1,), (0,)), ((), ())),
            preferred_element_type=jnp.float32)
        ssum = jnp.sum(qh.astype(jnp.float32) * ks, axis=1)
        m_acc[h, pl.ds(i * _QB, _QB)] = smax - ssum * (1.0 / L_K)

    @pl.when(i == pl.num_programs(0) - 1)
    def _topk():
        m = m_acc[...]
        L = m.shape[1]
        lanes = jax.lax.broadcasted_iota(jnp.int32, (H, L), 1)
        big = jnp.int32(L)
        vmin = jnp.min(m, axis=1, keepdims=True)
        minpos = jnp.min(jnp.where(m == vmin, lanes, big), axis=1,
                         keepdims=True)
        lane_pad = jax.lax.broadcasted_iota(jnp.int32, (H, _PAD), 1)
        pv0 = jnp.broadcast_to(minpos, (H, _PAD)).astype(jnp.int32)

        def step(u, carry):
            mc, pv = carry
            vmax = jnp.max(mc, axis=1, keepdims=True)
            sp = jnp.min(jnp.where(mc == vmax, lanes, big), axis=1,
                         keepdims=True)
            pv = jnp.where(lane_pad == u, sp, pv)
            mc = jnp.where(lanes == sp, -jnp.inf, mc)
            return mc, pv

        _, pv = jax.lax.fori_loop(0, n_top, step, (m, pv0))
        pos_ref[...] = pv
        ii = jax.lax.broadcasted_iota(jnp.int32, (_PAD, _PAD), 0)
        jj = jax.lax.broadcasted_iota(jnp.int32, (_PAD, _PAD), 1)
        eye = (ii == jj).astype(jnp.float32)
        pvt = jax.lax.dot_general(
            eye, pv.astype(jnp.float32), (((1,), (1,)), ((), ())),
            preferred_element_type=jnp.float32, precision=_HIGHEST)
        post_ref[...] = pvt.astype(jnp.int32)


def _compute_topk(q4, k4, cnt, ninf, n_top):
    L, H, D = q4.shape
    return pl.pallas_call(
        functools.partial(_mstats_body, n_top),
        grid=(L // _QB,),
        in_specs=[
            pl.BlockSpec((_QB, H, D), lambda i: (i, 0, 0)),
            pl.BlockSpec((L, H, D), lambda i: (0, 0, 0)),
            pl.BlockSpec((_QB, L), lambda i: (i, 0)),
            pl.BlockSpec((_QB, L), lambda i: (i, 0)),
        ],
        out_specs=[
            pl.BlockSpec((H, _PAD), lambda i: (0, 0)),
            pl.BlockSpec((_PAD, H), lambda i: (0, 0)),
        ],
        out_shape=[
            jax.ShapeDtypeStruct((H, _PAD), jnp.int32),
            jax.ShapeDtypeStruct((_PAD, H), jnp.int32),
        ],
        scratch_shapes=[pltpu.VMEM((H, L), jnp.float32)],
    )(q4, k4, cnt, ninf)


def _gather_body(n_top, post_ref, q_ref, qred_ref):
    H, D = q_ref.shape[1], q_ref.shape[2]
    i = pl.program_id(0)

    @pl.when(i == 0)
    def _init():
        qred_ref[...] = jnp.zeros(qred_ref.shape, qred_ref.dtype)

    base = i * _QB
    lanes = jax.lax.broadcasted_iota(jnp.int32, (_PAD, _QB), 1)
    row_ok = jax.lax.broadcasted_iota(jnp.int32, (_PAD, _QB), 0) < n_top
    for h in range(H):
        pvt = post_ref[:, h:h + 1]
        oh = jnp.where(((pvt - base) == lanes) & row_ok, 1.0, 0.0)
        qr = jax.lax.dot_general(
            oh.astype(jnp.bfloat16), q_ref[:, h, :], (((1,), (0,)), ((), ())),
            preferred_element_type=jnp.float32)
        qred_ref[h] += qr


def _gather_qred(post, q4, n_top):
    L, H, D = q4.shape
    return pl.pallas_call(
        functools.partial(_gather_body, n_top),
        grid=(L // _QB,),
        in_specs=[
            pl.BlockSpec((_PAD, H), lambda i: (0, 0)),
            pl.BlockSpec((_QB, H, D), lambda i: (i, 0, 0)),
        ],
        out_specs=pl.BlockSpec((H, _PAD, D), lambda i: (0, 0, 0)),
        out_shape=jax.ShapeDtypeStruct((H, _PAD, D), jnp.float32),
    )(post, q4)


def _attend_body(n_top, post_ref, qred_ref, k_ref, v_ref, ctx_ref):
    L = k_ref.shape[0]
    D = k_ref.shape[2]
    H = post_ref.shape[1]
    i = pl.program_id(0)
    lanes = jax.lax.broadcasted_iota(jnp.int32, (_PAD, L), 1)
    row_ok = jax.lax.broadcasted_iota(jnp.int32, (_PAD, L), 0) < n_top
    ones_col = jnp.ones((_PAD, 1), jnp.float32)
    post = post_ref[...].astype(jnp.float32)
    hsel = jax.lax.broadcasted_iota(jnp.int32, (H, 1), 0)
    for j in range(_HB):
        colsel = (hsel == (i * _HB + j)).astype(jnp.float32)
        pvt = jax.lax.dot_general(
            post, colsel, (((1,), (0,)), ((), ())),
            preferred_element_type=jnp.float32,
            precision=_HIGHEST).astype(jnp.int32)
        oh = jnp.where((pvt == lanes) & row_ok, 1.0, 0.0)
        k = k_ref[:, j, :]
        v = v_ref[:, j, :]
        qr = qred_ref[j].astype(jnp.bfloat16)
        qk = jax.lax.dot_general(
            qr, k, (((1,), (1,)), ((), ())),
            preferred_element_type=jnp.float32)
        s = qk * (1.0 / math.sqrt(D))
        smax = jnp.max(s, axis=1, keepdims=True)
        e = jnp.exp(s - smax)
        attn = e / jnp.sum(e, axis=1, keepdims=True)
        upd = jax.lax.dot_general(
            attn.astype(jnp.bfloat16), v, (((1,), (0,)), ((), ())),
            preferred_element_type=jnp.float32)
        vmean = jnp.sum(v.astype(jnp.float32), axis=0,
                        keepdims=True) * (1.0 / L)

        ind_col = jax.lax.dot_general(
            oh, ones_col, (((0,), (0,)), ((), ())),
            preferred_element_type=jnp.float32, precision=_HIGHEST)
        scattered = jax.lax.dot_general(
            oh, upd, (((0,), (0,)), ((), ())),
            preferred_element_type=jnp.float32, precision=_HIGHEST)
        ctx_ref[:, j, :] = scattered + (1.0 - ind_col) * vmean


def _attend(post, qred, k4, v4, n_top):
    L, H, D = k4.shape
    return pl.pallas_call(
        functools.partial(_attend_body, n_top),
        grid=(H // _HB,),
        in_specs=[
            pl.BlockSpec((_PAD, H), lambda i: (0, 0)),
            pl.BlockSpec((_HB, _PAD, D), lambda i: (i, 0, 0)),
            pl.BlockSpec((L, _HB, D), lambda i: (0, i, 0)),
            pl.BlockSpec((L, _HB, D), lambda i: (0, i, 0)),
        ],
        out_specs=pl.BlockSpec((L, _HB, D), lambda i: (0, i, 0)),
        out_shape=jax.ShapeDtypeStruct((L, H, D), jnp.float32),
        compiler_params=pltpu.CompilerParams(
            vmem_limit_bytes=64 * 1024 * 1024),
    )(post, qred, k4, v4)


def kernel(queries, keys, values):
    B, L_Q, H, D = queries.shape
    L_K = keys.shape[1]
    assert B == 1
    sample_k = max(1, min(_FACTOR * int(np.ceil(np.log(L_Q))), L_K))
    n_top = max(1, min(_FACTOR * int(np.ceil(np.log(L_K))), L_Q))
    assert n_top <= _PAD

    q4 = queries[0].astype(jnp.bfloat16)
    k4 = keys[0].astype(jnp.bfloat16)
    v4 = values[0].astype(jnp.bfloat16)

    cnt, ninf = _count_table(L_Q, L_K, sample_k)
    pos, post = _compute_topk(q4, k4, cnt, ninf, n_top)
    del pos
    qred = _gather_qred(post, q4, n_top)
    ctx = _attend(post, qred, k4, v4, n_top)
    return ctx[None]


# speedup vs baseline: 1.1181x; 1.1181x over previous
import functools
import math

import jax
import jax.numpy as jnp
import numpy as np
from jax.experimental import pallas as pl
from jax.experimental.pallas import tpu as pltpu

_FACTOR = 5
_PAD = 64
_HIGHEST = jax.lax.Precision.HIGHEST
_QB = 256


def _threefry2x32(k0, k1, x0, x1):
    def rotl(x, d):
        return (x << np.uint32(d)) | (x >> np.uint32(32 - d))

    ks0, ks1 = np.uint32(k0), np.uint32(k1)
    ks2 = ks0 ^ ks1 ^ np.uint32(0x1BD11BDA)
    x0 = x0 + ks0
    x1 = x1 + ks1
    rot1, rot2 = (13, 15, 26, 6), (17, 29, 16, 24)
    sched = [(rot1, ks1, ks2, 1), (rot2, ks2, ks0, 2), (rot1, ks0, ks1, 3),
             (rot2, ks1, ks2, 4), (rot1, ks2, ks0, 5)]
    for rots, a, b, i in sched:
        for r in rots:
            x0 = x0 + x1
            x1 = rotl(x1, r)
            x1 = x1 ^ x0
        x0 = x0 + a
        x1 = x1 + b + np.uint32(i)
    return x0, x1


def _tf_random_bits(k, n):
    i = np.arange(n, dtype=np.uint64)
    hi = (i >> np.uint64(32)).astype(np.uint32)
    lo = (i & np.uint64(0xFFFFFFFF)).astype(np.uint32)
    b1, b2 = _threefry2x32(k[0], k[1], hi, lo)
    return b1 ^ b2


@functools.lru_cache(maxsize=None)
def _count_table(L_Q: int, L_K: int, sample_k: int):
    assert L_K & (L_K - 1) == 0
    old = np.seterr(over="ignore")
    try:
        key = (np.uint32(0), np.uint32(42))
        hi = np.zeros(2, np.uint32)
        lo = np.arange(2, dtype=np.uint32)
        b1, b2 = _threefry2x32(key[0], key[1], hi, lo)
        k2 = (b1[1], b2[1])
        bits = _tf_random_bits(k2, L_Q * sample_k).reshape(L_Q, sample_k)
        idx = (bits % np.uint32(L_K)).astype(np.int32)
    finally:
        np.seterr(**old)
    cnt = np.zeros((L_Q, L_K), np.float32)
    np.add.at(cnt, (np.arange(L_Q)[:, None], idx), 1.0)
    return cnt


def _mstats_body(n_top, q_ref, k_ref, cnt_ref, pos_ref, m_acc):
    H = q_ref.shape[1]
    L_K = k_ref.shape[0]
    i = pl.program_id(0)
    cnt = cnt_ref[...]
    mask = cnt > 0.0
    for h in range(H):
        s = jax.lax.dot_general(
            q_ref[:, h, :].astype(jnp.bfloat16),
            k_ref[:, h, :].astype(jnp.bfloat16),
            (((1,), (1,)), ((), ())),
            preferred_element_type=jnp.float32)
        smax = jnp.max(jnp.where(mask, s, -jnp.inf), axis=1)
        ssum = jnp.sum(s * cnt, axis=1)
        m_acc[h, pl.ds(i * _QB, _QB)] = smax - ssum * (1.0 / L_K)

    @pl.when(i == pl.num_programs(0) - 1)
    def _topk():
        m = m_acc[...]
        L = m.shape[1]
        lanes = jax.lax.broadcasted_iota(jnp.int32, (H, L), 1)
        big = jnp.int32(L)
        vmin = jnp.min(m, axis=1, keepdims=True)
        minpos = jnp.min(jnp.where(m == vmin, lanes, big), axis=1,
                         keepdims=True)
        lane_pad = jax.lax.broadcasted_iota(jnp.int32, (H, _PAD), 1)
        pv0 = jnp.broadcast_to(minpos, (H, _PAD)).astype(jnp.int32)

        def step(u, carry):
            mc, pv = carry
            vmax = jnp.max(mc, axis=1, keepdims=True)
            sp = jnp.min(jnp.where(mc == vmax, lanes, big), axis=1,
                         keepdims=True)
            pv = jnp.where(lane_pad == u, sp, pv)
            mc = jnp.where(lanes == sp, -jnp.inf, mc)
            return mc, pv

        _, pv = jax.lax.fori_loop(0, n_top, step, (m, pv0))
        pos_ref[...] = pv


def _compute_topk(q4, k4, cnt, n_top):
    L, H, D = q4.shape
    return pl.pallas_call(
        functools.partial(_mstats_body, n_top),
        grid=(L // _QB,),
        in_specs=[
            pl.BlockSpec((_QB, H, D), lambda i: (i, 0, 0)),
            pl.BlockSpec((L, H, D), lambda i: (0, 0, 0)),
            pl.BlockSpec((_QB, L), lambda i: (i, 0)),
        ],
        out_specs=pl.BlockSpec((H, _PAD), lambda i: (0, 0)),
        out_shape=jax.ShapeDtypeStruct((H, _PAD), jnp.int32),
        scratch_shapes=[pltpu.VMEM((H, L), jnp.float32)],
    )(q4, k4, cnt)


def _attend_body(n_top, pos_ref, q_ref, k_ref, v_ref, ctx_ref):
    L, D = k_ref.shape[0], k_ref.shape[1]
    posvec = pos_ref[0]

    ii = jax.lax.broadcasted_iota(jnp.int32, (_PAD, _PAD), 0)
    jj = jax.lax.broadcasted_iota(jnp.int32, (_PAD, _PAD), 1)
    eye = (ii == jj).astype(jnp.float32)
    pv_col = jax.lax.dot_general(
        eye, posvec.astype(jnp.float32), (((1,), (1,)), ((), ())),
        preferred_element_type=jnp.float32, precision=_HIGHEST)
    pv_col_i = pv_col.astype(jnp.int32)
    lanes_i = jax.lax.broadcasted_iota(jnp.int32, (_PAD, L), 1)
    row_ok = jax.lax.broadcasted_iota(jnp.int32, (_PAD, L), 0) < n_top
    oh = jnp.where((pv_col_i == lanes_i) & row_ok, 1.0, 0.0)

    q = q_ref[...]
    k = k_ref[...]
    v = v_ref[...]
    qr = jax.lax.dot_general(oh.astype(jnp.bfloat16), q.astype(jnp.bfloat16),
                             (((1,), (0,)), ((), ())),
                             preferred_element_type=jnp.float32)
    qk = jax.lax.dot_general(qr.astype(jnp.bfloat16), k.astype(jnp.bfloat16),
                             (((1,), (1,)), ((), ())),
                             preferred_element_type=jnp.float32)
    s = qk * (1.0 / math.sqrt(D))
    smax = jnp.max(s, axis=1, keepdims=True)
    e = jnp.exp(s - smax)
    attn = e / jnp.sum(e, axis=1, keepdims=True)
    upd = jax.lax.dot_general(attn.astype(jnp.bfloat16),
                              v.astype(jnp.bfloat16),
                              (((1,), (0,)), ((), ())),
                              preferred_element_type=jnp.float32)
    vmean = jnp.sum(v, axis=0, keepdims=True) * (1.0 / L)

    ones_col = jnp.ones((_PAD, 1), jnp.float32)
    ind_col = jax.lax.dot_general(oh, ones_col, (((0,), (0,)), ((), ())),
                                  preferred_element_type=jnp.float32,
                                  precision=_HIGHEST)
    scattered = jax.lax.dot_general(oh, upd, (((0,), (0,)), ((), ())),
                                    preferred_element_type=jnp.float32,
                                    precision=_HIGHEST)
    ctx_ref[...] = scattered + (1.0 - ind_col) * vmean


def _attend(pos, q4, k4, v4, n_top):
    L, H, D = q4.shape
    pos3 = pos.reshape(H, 1, _PAD)
    q2 = q4.reshape(L, H * D)
    k2 = k4.reshape(L, H * D)
    v2 = v4.reshape(L, H * D)
    return pl.pallas_call(
        functools.partial(_attend_body, n_top),
        grid=(H,),
        in_specs=[
            pl.BlockSpec((1, 1, _PAD), lambda h: (h, 0, 0)),
            pl.BlockSpec((L, D), lambda h: (0, h)),
            pl.BlockSpec((L, D), lambda h: (0, h)),
            pl.BlockSpec((L, D), lambda h: (0, h)),
        ],
        out_specs=pl.BlockSpec((L, D), lambda h: (0, h)),
        out_shape=jax.ShapeDtypeStruct((L, H * D), jnp.float32),
    )(pos3, q2, k2, v2)


def kernel(queries, keys, values):
    B, L_Q, H, D = queries.shape
    L_K = keys.shape[1]
    assert B == 1
    sample_k = max(1, min(_FACTOR * int(np.ceil(np.log(L_Q))), L_K))
    n_top = max(1, min(_FACTOR * int(np.ceil(np.log(L_K))), L_Q))
    assert n_top <= _PAD

    q4 = queries[0]
    k4 = keys[0]
    v4 = values[0]

    cnt = _count_table(L_Q, L_K, sample_k)
    pos = _compute_topk(q4, k4, cnt, n_top)
    ctx2d = _attend(pos, q4, k4, v4, n_top)
    return ctx2d.reshape(1, L_Q, H, D)
